# baseline (speedup 1.0000x reference)
import jax
import jax.numpy as jnp
from jax.experimental import pallas as pl

_BT = 512


def _add_kernel(x_ref, pos_ref, o_ref):
    o_ref[...] = x_ref[...] + pos_ref[...][None, :, :]


def kernel(x, pos_embedding):
    B, T, E = x.shape
    grid = (T // _BT,)
    return pl.pallas_call(
        _add_kernel,
        grid=grid,
        in_specs=[
            pl.BlockSpec((B, _BT, E), lambda t: (0, t, 0)),
            pl.BlockSpec((_BT, E), lambda t: (t, 0)),
        ],
        out_specs=pl.BlockSpec((B, _BT, E), lambda t: (0, t, 0)),
        out_shape=jax.ShapeDtypeStruct((B, T, E), x.dtype),
    )(x, pos_embedding)
